# baseline (speedup 1.0000x reference)
import jax
import jax.numpy as jnp
from jax import lax
from jax.experimental import pallas as pl
from jax.experimental.pallas import tpu as pltpu
from jax.experimental.pallas import tpu_sc as plsc

N = 10000
E = 320000
D = 128
D2 = 64
NP = 10112
G = 128
NC = 2
NS = 16
KG_PROP = 160
KG_DEG = 80
EP = NC * NS * KG_DEG * G
NGRP = EP // G
RPT = NP // NS
RCH = 79

_mesh = plsc.VectorSubcoreMesh(core_axis_name="c", subcore_axis_name="s")
_sc_params = pltpu.CompilerParams(use_tc_tiling_on_sc=False)


def _deg_body(col_hbm, zeros_hbm, ones_hbm, out_hbm, cbuf, obuf, acc):
    c = lax.axis_index("c")
    s = lax.axis_index("s")
    w = s * NC + c
    pltpu.sync_copy(col_hbm.at[pl.ds(w * KG_DEG, KG_DEG)], cbuf)
    pltpu.sync_copy(ones_hbm, obuf)
    pltpu.sync_copy(zeros_hbm.at[pl.ds(s * RPT, RPT)], acc.at[pl.ds(s * RPT, RPT)])
    plsc.subcore_barrier()

    def step(g):
        pltpu.sync_copy(obuf, acc.at[cbuf.at[g]], add=True)

    lax.fori_loop(0, KG_DEG, lambda g, _: (step(g), 0)[1], 0)
    plsc.subcore_barrier()
    pltpu.sync_copy(acc.at[pl.ds(s * RPT, RPT)], out_hbm.at[c, pl.ds(s * RPT, RPT)])


_deg_call = pl.kernel(
    _deg_body,
    out_type=jax.ShapeDtypeStruct((NC, NP, 16), jnp.float32),
    mesh=_mesh,
    scratch_types=[
        pltpu.VMEM((KG_DEG, G), jnp.int32),
        pltpu.VMEM((G, 16), jnp.float32),
        pltpu.VMEM_SHARED((NP, 16), jnp.float32),
    ],
    compiler_params=_sc_params,
)


RING = 4
SLOTS = 2 * RING
NQ = 5
QKG = KG_PROP // NQ
NROUND = QKG // SLOTS


def _prop_body(u_hbm, row_hbm, col_hbm, p_hbm, out_hbm,
               rbuf, cbuf, gbuf, acc, pbuf, obuf, gsem, ssem, isem):
    c = lax.axis_index("c")
    s = lax.axis_index("s")
    base = s * KG_PROP

    def _load_idx(q):
        sl = q % 2
        pltpu.async_copy(
            row_hbm.at[c, pl.ds(base + q * QKG, QKG)], rbuf.at[sl], isem.at[0]
        )
        pltpu.async_copy(
            col_hbm.at[pl.ds(base + q * QKG, QKG)], cbuf.at[sl], isem.at[1]
        )

    def _wait_idx(q):
        sl = q % 2
        pltpu.make_async_copy(
            row_hbm.at[c, pl.ds(base, QKG)], rbuf.at[sl], isem.at[0]
        ).wait()
        pltpu.make_async_copy(
            col_hbm.at[pl.ds(base, QKG)], cbuf.at[sl], isem.at[1]
        ).wait()

    _load_idx(0)
    pltpu.sync_copy(
        u_hbm.at[pl.ds(c * NP + s * RPT, RPT)], acc.at[pl.ds(s * RPT, RPT)]
    )
    _wait_idx(0)
    plsc.subcore_barrier()

    for q in range(NQ):
        rb = rbuf.at[q % 2]
        cb = cbuf.at[q % 2]
        if q + 1 < NQ:
            _load_idx(q + 1)

        def _gfire(b, g):
            pltpu.async_copy(u_hbm.at[rb.at[g]], gbuf.at[b], gsem.at[b])

        def _gdrain(b, g):
            pltpu.make_async_copy(u_hbm.at[rb.at[g]], gbuf.at[b], gsem.at[b]).wait()

        def _sfire(b, g):
            pltpu.async_copy(gbuf.at[b], acc.at[cb.at[g]], ssem.at[b], add=True)

        def _swait(b):
            pltpu.make_async_copy(gbuf.at[b], acc.at[cb.at[0]], ssem.at[b]).wait()

        for b in range(RING):
            _gfire(b, b)
        for g in range(SLOTS):
            _gdrain(g, g)
            _sfire(g, g)
            if g >= RING:
                _swait((g + RING) % SLOTS)
            _gfire((g + RING) % SLOTS, g + RING)

        def mid_round(i, _):
            g0 = i * SLOTS
            for b in range(SLOTS):
                g = g0 + b
                _gdrain(b, g)
                _sfire(b, g)
                _swait((b + RING) % SLOTS)
                _gfire((b + RING) % SLOTS, g + RING)
            return 0

        lax.fori_loop(1, NROUND - 1, mid_round, 0)
        for b in range(SLOTS):
            g = (NROUND - 1) * SLOTS + b
            _gdrain(b, g)
            _sfire(b, g)
            _swait((b + RING) % SLOTS)
            if g + RING < QKG:
                _gfire((g + RING) % SLOTS, g + RING)
        for b in range(RING):
            _swait((b + RING) % SLOTS)
        if q + 1 < NQ:
            _wait_idx(q + 1)

    plsc.subcore_barrier()
    for k in range(RPT // RCH):
        r0 = s * RPT + k * RCH
        pltpu.sync_copy(p_hbm.at[pl.ds(r0, RCH)], pbuf)
        pltpu.sync_copy(acc.at[pl.ds(r0, RCH)], obuf)

        def scale_row(i, _):
            pv = pbuf[i]
            for j in range(D2 // 16):
                obuf[i, pl.ds(j * 16, 16)] = obuf[i, pl.ds(j * 16, 16)] * pv
            return 0

        lax.fori_loop(0, RCH, scale_row, 0)
        pltpu.sync_copy(obuf, out_hbm.at[c, pl.ds(r0, RCH)])


_prop_call = pl.kernel(
    _prop_body,
    out_type=jax.ShapeDtypeStruct((NC, NP, D2), jnp.float32),
    mesh=_mesh,
    scratch_types=[
        pltpu.VMEM((2, QKG, G), jnp.int32),
        pltpu.VMEM((2, QKG, G), jnp.int32),
        pltpu.VMEM((SLOTS, G, D2), jnp.float32),
        pltpu.VMEM_SHARED((NP, D2), jnp.float32),
        pltpu.VMEM((RCH, 16), jnp.float32),
        pltpu.VMEM((RCH, D2), jnp.float32),
        pltpu.SemaphoreType.DMA((SLOTS,)),
        pltpu.SemaphoreType.DMA((SLOTS,)),
        pltpu.SemaphoreType.DMA((2,)),
    ],
    compiler_params=_sc_params,
)


def _pre_body(x_ref, dg_ref, u_ref, dinv_ref, dinvw_ref, dinv2w_ref):
    deg = dg_ref[0, :, 0:1] + dg_ref[1, :, 0:1] + 1.0
    dinv = jnp.where(deg > 0, lax.rsqrt(deg), 0.0)
    dinv_ref[...] = dinv
    dinvw_ref[...] = jnp.broadcast_to(dinv, (NP, 16))
    dinv2w_ref[...] = jnp.broadcast_to(dinv * dinv, (NP, 16))
    u = x_ref[...] * dinv[:N]
    u_ref[0, :N, :] = u[:, :D2]
    u_ref[1, :N, :] = u[:, D2:]
    u_ref[0, N:, :] = jnp.zeros((NP - N, D2), jnp.float32)
    u_ref[1, N:, :] = jnp.zeros((NP - N, D2), jnp.float32)


def _mm1_body(w_ref, dinv_ref, w1_ref, b1_ref, o_ref):
    full = jnp.concatenate([w_ref[0], w_ref[1]], axis=1)
    y = jnp.dot(full, w1_ref[...], preferred_element_type=jnp.float32)
    h = jnp.maximum(y + b1_ref[...], 0.0) * dinv_ref[...]
    o_ref[0] = h[:, :D2]
    o_ref[1] = h[:, D2:]


def _mm2_body(w_ref, w2_ref, b2_ref, o_ref):
    full = jnp.concatenate([w_ref[0][:N], w_ref[1][:N]], axis=1)
    o_ref[...] = jnp.dot(full, w2_ref[...],
                         preferred_element_type=jnp.float32) + b2_ref[...]


_pre_call = pl.pallas_call(
    _pre_body,
    out_shape=[
        jax.ShapeDtypeStruct((NC, NP, D2), jnp.float32),
        jax.ShapeDtypeStruct((NP, 1), jnp.float32),
        jax.ShapeDtypeStruct((NP, 16), jnp.float32),
        jax.ShapeDtypeStruct((NP, 16), jnp.float32),
    ],
)

_mm1_call = pl.pallas_call(
    _mm1_body,
    out_shape=jax.ShapeDtypeStruct((NC, NP, D2), jnp.float32),
)

_mm2_call = pl.pallas_call(
    _mm2_body,
    out_shape=jax.ShapeDtypeStruct((N, D), jnp.float32),
)


@jax.jit
def kernel(x, edge_index, W1, b1, W2, b2):
    row = edge_index[0].astype(jnp.int32)
    col = edge_index[1].astype(jnp.int32)
    pad = EP - E
    rowp = jnp.concatenate([row, jnp.zeros((pad,), jnp.int32)])
    colp = jnp.concatenate([col, jnp.full((pad,), N, jnp.int32)])
    row2 = jnp.stack([rowp, rowp + NP]).reshape(NC, NGRP, G)
    colg = colp.reshape(NGRP, G)

    zeros16 = jnp.zeros((NP, 16), jnp.float32)
    ones16 = jnp.ones((G, 16), jnp.float32)

    deg16 = _deg_call(colg, zeros16, ones16)
    u, dinv, dinvw, dinv2w = _pre_call(x, deg16)

    u2 = _prop_call(u.reshape(NC * NP, D2), row2, colg, dinv2w)
    pm1 = _prop_call(u2.reshape(NC * NP, D2), row2, colg, dinvw)
    u3 = _mm1_call(pm1, dinv, W1, b1.reshape(1, D))
    u4 = _prop_call(u3.reshape(NC * NP, D2), row2, colg, dinv2w)
    pm2 = _prop_call(u4.reshape(NC * NP, D2), row2, colg, dinvw)
    return _mm2_call(pm2, W2, b2.reshape(1, D))


# speedup vs baseline: 1.0491x; 1.0491x over previous
import jax
import jax.numpy as jnp
from jax import lax
from jax.experimental import pallas as pl
from jax.experimental.pallas import tpu as pltpu
from jax.experimental.pallas import tpu_sc as plsc

N = 10000
E = 320000
D = 128
D2 = 64
NP = 10112
G = 128
NC = 2
NS = 16
KG_PROP = 160
KG_DEG = 80
EP = NC * NS * KG_DEG * G
NGRP = EP // G
RPT = NP // NS

_mesh = plsc.VectorSubcoreMesh(core_axis_name="c", subcore_axis_name="s")
_sc_params = pltpu.CompilerParams(use_tc_tiling_on_sc=False)


def _deg_body(col_hbm, zeros_hbm, ones_hbm, out_hbm, cbuf, obuf, acc):
    c = lax.axis_index("c")
    s = lax.axis_index("s")
    w = s * NC + c
    pltpu.sync_copy(col_hbm.at[pl.ds(w * KG_DEG, KG_DEG)], cbuf)
    pltpu.sync_copy(ones_hbm, obuf)
    pltpu.sync_copy(zeros_hbm.at[pl.ds(s * RPT, RPT)], acc.at[pl.ds(s * RPT, RPT)])
    plsc.subcore_barrier()

    def step(g):
        pltpu.sync_copy(obuf, acc.at[cbuf.at[g]], add=True)

    lax.fori_loop(0, KG_DEG, lambda g, _: (step(g), 0)[1], 0)
    plsc.subcore_barrier()
    pltpu.sync_copy(acc.at[pl.ds(s * RPT, RPT)], out_hbm.at[c, pl.ds(s * RPT, RPT)])


_deg_call = pl.kernel(
    _deg_body,
    out_type=jax.ShapeDtypeStruct((NC, NP, 16), jnp.float32),
    mesh=_mesh,
    scratch_types=[
        pltpu.VMEM((KG_DEG, G), jnp.int32),
        pltpu.VMEM((G, 16), jnp.float32),
        pltpu.VMEM_SHARED((NP, 16), jnp.float32),
    ],
    compiler_params=_sc_params,
)


RING = 4


def _prop_body(u_hbm, row_hbm, col_hbm, out_hbm, rbuf, cbuf, gbuf, acc, sem):
    c = lax.axis_index("c")
    s = lax.axis_index("s")
    pltpu.sync_copy(row_hbm.at[c, pl.ds(s * KG_PROP, KG_PROP)], rbuf)
    pltpu.sync_copy(col_hbm.at[pl.ds(s * KG_PROP, KG_PROP)], cbuf)
    pltpu.sync_copy(
        u_hbm.at[pl.ds(c * NP + s * RPT, RPT)], acc.at[pl.ds(s * RPT, RPT)]
    )
    plsc.subcore_barrier()

    def _fire(b, g):
        pltpu.async_copy(u_hbm.at[rbuf.at[g]], gbuf.at[b], sem.at[b])

    def _drain(b, g):
        pltpu.make_async_copy(u_hbm.at[rbuf.at[g]], gbuf.at[b], sem.at[b]).wait()

    for b in range(RING):
        _fire(b, b)

    def outer(i, _):
        g0 = i * RING
        for b in range(RING):
            g = g0 + b
            _drain(b, g)
            pltpu.sync_copy(gbuf.at[b], acc.at[cbuf.at[g]], add=True)
            _fire(b, g + RING)
        return 0

    lax.fori_loop(0, KG_PROP // RING - 1, outer, 0)
    for b in range(RING):
        g = KG_PROP - RING + b
        _drain(b, g)
        pltpu.sync_copy(gbuf.at[b], acc.at[cbuf.at[g]], add=True)

    plsc.subcore_barrier()
    pltpu.sync_copy(acc.at[pl.ds(s * RPT, RPT)], out_hbm.at[c, pl.ds(s * RPT, RPT)])


_prop_call = pl.kernel(
    _prop_body,
    out_type=jax.ShapeDtypeStruct((NC, NP, D2), jnp.float32),
    mesh=_mesh,
    scratch_types=[
        pltpu.VMEM((KG_PROP, G), jnp.int32),
        pltpu.VMEM((KG_PROP, G), jnp.int32),
        pltpu.VMEM((RING, G, D2), jnp.float32),
        pltpu.VMEM_SHARED((NP, D2), jnp.float32),
        pltpu.SemaphoreType.DMA((RING,)),
    ],
    compiler_params=_sc_params,
)


def _pre_body(x_ref, dg_ref, u_ref, dinv_ref, dinv2_ref):
    deg = dg_ref[0, :, 0:1] + dg_ref[1, :, 0:1] + 1.0
    dinv = jnp.where(deg > 0, lax.rsqrt(deg), 0.0)
    dinv_ref[...] = dinv
    dinv2_ref[...] = dinv * dinv
    u = x_ref[...] * dinv[:N]
    u_ref[0, :N, :] = u[:, :D2]
    u_ref[1, :N, :] = u[:, D2:]
    u_ref[0, N:, :] = jnp.zeros((NP - N, D2), jnp.float32)
    u_ref[1, N:, :] = jnp.zeros((NP - N, D2), jnp.float32)


def _mid_body(v_ref, dinv2_ref, o_ref):
    o_ref[0] = v_ref[0] * dinv2_ref[...]
    o_ref[1] = v_ref[1] * dinv2_ref[...]


def _mm1_body(w_ref, dinv_ref, w1_ref, b1_ref, o_ref):
    full = jnp.concatenate([w_ref[0], w_ref[1]], axis=1)
    y = jnp.dot(full * dinv_ref[...], w1_ref[...],
                preferred_element_type=jnp.float32) + b1_ref[...]
    h = jnp.maximum(y, 0.0) * dinv_ref[...]
    o_ref[0] = h[:, :D2]
    o_ref[1] = h[:, D2:]


def _mm2_body(w_ref, dinv_ref, w2_ref, b2_ref, o_ref):
    full = jnp.concatenate([w_ref[0][:N], w_ref[1][:N]], axis=1)
    o_ref[...] = jnp.dot(full * dinv_ref[:N], w2_ref[...],
                         preferred_element_type=jnp.float32) + b2_ref[...]


_pre_call = pl.pallas_call(
    _pre_body,
    out_shape=[
        jax.ShapeDtypeStruct((NC, NP, D2), jnp.float32),
        jax.ShapeDtypeStruct((NP, 1), jnp.float32),
        jax.ShapeDtypeStruct((NP, 1), jnp.float32),
    ],
)

_mid_call = pl.pallas_call(
    _mid_body,
    out_shape=jax.ShapeDtypeStruct((NC, NP, D2), jnp.float32),
)

_mm1_call = pl.pallas_call(
    _mm1_body,
    out_shape=jax.ShapeDtypeStruct((NC, NP, D2), jnp.float32),
)

_mm2_call = pl.pallas_call(
    _mm2_body,
    out_shape=jax.ShapeDtypeStruct((N, D), jnp.float32),
)


@jax.jit
def kernel(x, edge_index, W1, b1, W2, b2):
    row = edge_index[0].astype(jnp.int32)
    col = edge_index[1].astype(jnp.int32)
    pad = EP - E
    rowp = jnp.concatenate([row, jnp.zeros((pad,), jnp.int32)])
    colp = jnp.concatenate([col, jnp.full((pad,), N, jnp.int32)])
    row2 = jnp.stack([rowp, rowp + NP]).reshape(NC, NGRP, G)
    colg = colp.reshape(NGRP, G)

    zeros16 = jnp.zeros((NP, 16), jnp.float32)
    ones16 = jnp.ones((G, 16), jnp.float32)

    deg16 = _deg_call(colg, zeros16, ones16)
    u, dinv, dinv2 = _pre_call(x, deg16)

    v1 = _prop_call(u.reshape(NC * NP, D2), row2, colg)
    u2 = _mid_call(v1, dinv2)
    w1 = _prop_call(u2.reshape(NC * NP, D2), row2, colg)
    u3 = _mm1_call(w1, dinv, W1, b1.reshape(1, D))
    v2 = _prop_call(u3.reshape(NC * NP, D2), row2, colg)
    u4 = _mid_call(v2, dinv2)
    w2 = _prop_call(u4.reshape(NC * NP, D2), row2, colg)
    return _mm2_call(w2, dinv, W2, b2.reshape(1, D))


# speedup vs baseline: 1.0800x; 1.0294x over previous
import jax
import jax.numpy as jnp
from jax import lax
from jax.experimental import pallas as pl
from jax.experimental.pallas import tpu as pltpu
from jax.experimental.pallas import tpu_sc as plsc

N = 10000
E = 320000
D = 128
D2 = 64
NP = 10112
G = 128
NC = 2
NS = 16
KG_PROP = 160
KG_DEG = 80
EP = NC * NS * KG_DEG * G
NGRP = EP // G
RPT = NP // NS

_mesh = plsc.VectorSubcoreMesh(core_axis_name="c", subcore_axis_name="s")
_sc_params = pltpu.CompilerParams(use_tc_tiling_on_sc=False)


def _deg_body(col_hbm, zeros_hbm, ones_hbm, out_hbm, cbuf, obuf, acc):
    c = lax.axis_index("c")
    s = lax.axis_index("s")
    w = s * NC + c
    pltpu.sync_copy(col_hbm.at[pl.ds(w * KG_DEG, KG_DEG)], cbuf)
    pltpu.sync_copy(ones_hbm, obuf)
    pltpu.sync_copy(zeros_hbm.at[pl.ds(s * RPT, RPT)], acc.at[pl.ds(s * RPT, RPT)])
    plsc.subcore_barrier()

    def step(g):
        pltpu.sync_copy(obuf, acc.at[cbuf.at[g]], add=True)

    lax.fori_loop(0, KG_DEG, lambda g, _: (step(g), 0)[1], 0)
    plsc.subcore_barrier()
    pltpu.sync_copy(acc.at[pl.ds(s * RPT, RPT)], out_hbm.at[c, pl.ds(s * RPT, RPT)])


_deg_call = pl.kernel(
    _deg_body,
    out_type=jax.ShapeDtypeStruct((NC, NP, 8), jnp.float32),
    mesh=_mesh,
    scratch_types=[
        pltpu.VMEM((KG_DEG, G), jnp.int32),
        pltpu.VMEM((G, 8), jnp.float32),
        pltpu.VMEM_SHARED((NP, 8), jnp.float32),
    ],
    compiler_params=_sc_params,
)


RING = 4


def _prop_body(u_hbm, row_hbm, col_hbm, out_hbm, rbuf, cbuf, gbuf, acc, sem):
    c = lax.axis_index("c")
    s = lax.axis_index("s")
    uc = u_hbm.at[c]
    pltpu.sync_copy(row_hbm.at[pl.ds(s * KG_PROP, KG_PROP)], rbuf)
    pltpu.sync_copy(col_hbm.at[pl.ds(s * KG_PROP, KG_PROP)], cbuf)
    pltpu.sync_copy(uc.at[pl.ds(s * RPT, RPT)], acc.at[pl.ds(s * RPT, RPT)])
    plsc.subcore_barrier()

    def _fire(b, g):
        pltpu.async_copy(uc.at[rbuf.at[g]], gbuf.at[b], sem.at[b])

    def _drain(b, g):
        pltpu.make_async_copy(uc.at[rbuf.at[g]], gbuf.at[b], sem.at[b]).wait()

    for b in range(RING):
        _fire(b, b)

    def outer(i, _):
        g0 = i * RING
        for b in range(RING):
            g = g0 + b
            _drain(b, g)
            pltpu.sync_copy(gbuf.at[b], acc.at[cbuf.at[g]], add=True)
            _fire(b, g + RING)
        return 0

    lax.fori_loop(0, KG_PROP // RING - 1, outer, 0)
    for b in range(RING):
        g = KG_PROP - RING + b
        _drain(b, g)
        pltpu.sync_copy(gbuf.at[b], acc.at[cbuf.at[g]], add=True)

    plsc.subcore_barrier()
    pltpu.sync_copy(acc.at[pl.ds(s * RPT, RPT)], out_hbm.at[c, pl.ds(s * RPT, RPT)])


_prop_call = pl.kernel(
    _prop_body,
    out_type=jax.ShapeDtypeStruct((NC, NP, D2), jnp.float32),
    mesh=_mesh,
    scratch_types=[
        pltpu.VMEM((KG_PROP, G), jnp.int32),
        pltpu.VMEM((KG_PROP, G), jnp.int32),
        pltpu.VMEM((RING, G, D2), jnp.float32),
        pltpu.VMEM_SHARED((NP, D2), jnp.float32),
        pltpu.SemaphoreType.DMA((RING,)),
    ],
    compiler_params=_sc_params,
)


def _pre_body(x_ref, dg_ref, u_ref, dinv_ref, dinv2_ref):
    deg = dg_ref[0, :, 0:1] + dg_ref[1, :, 0:1] + 1.0
    dinv = jnp.where(deg > 0, lax.rsqrt(deg), 0.0)
    dinv_ref[...] = dinv
    dinv2_ref[...] = dinv * dinv
    u = x_ref[...] * dinv[:N]
    u_ref[0, :N, :] = u[:, :D2]
    u_ref[1, :N, :] = u[:, D2:]
    u_ref[0, N:, :] = jnp.zeros((NP - N, D2), jnp.float32)
    u_ref[1, N:, :] = jnp.zeros((NP - N, D2), jnp.float32)


def _mid_body(v_ref, dinv2_ref, o_ref):
    o_ref[0] = v_ref[0] * dinv2_ref[...]
    o_ref[1] = v_ref[1] * dinv2_ref[...]


def _mm1_body(w_ref, dinv_ref, w1_ref, b1_ref, o_ref):
    full = jnp.concatenate([w_ref[0], w_ref[1]], axis=1)
    y = jnp.dot(full * dinv_ref[...], w1_ref[...],
                preferred_element_type=jnp.float32) + b1_ref[...]
    h = jnp.maximum(y, 0.0) * dinv_ref[...]
    o_ref[0] = h[:, :D2]
    o_ref[1] = h[:, D2:]


def _mm2_body(w_ref, dinv_ref, w2_ref, b2_ref, o_ref):
    full = jnp.concatenate([w_ref[0][:N], w_ref[1][:N]], axis=1)
    o_ref[...] = jnp.dot(full * dinv_ref[:N], w2_ref[...],
                         preferred_element_type=jnp.float32) + b2_ref[...]


_pre_call = pl.pallas_call(
    _pre_body,
    out_shape=[
        jax.ShapeDtypeStruct((NC, NP, D2), jnp.float32),
        jax.ShapeDtypeStruct((NP, 1), jnp.float32),
        jax.ShapeDtypeStruct((NP, 1), jnp.float32),
    ],
)

_mid_call = pl.pallas_call(
    _mid_body,
    out_shape=jax.ShapeDtypeStruct((NC, NP, D2), jnp.float32),
)

_mm1_call = pl.pallas_call(
    _mm1_body,
    out_shape=jax.ShapeDtypeStruct((NC, NP, D2), jnp.float32),
)

_mm2_call = pl.pallas_call(
    _mm2_body,
    out_shape=jax.ShapeDtypeStruct((N, D), jnp.float32),
)


@jax.jit
def kernel(x, edge_index, W1, b1, W2, b2):
    row = edge_index[0].astype(jnp.int32)
    col = edge_index[1].astype(jnp.int32)
    pad = EP - E
    rowp = jnp.concatenate([row, jnp.zeros((pad,), jnp.int32)])
    colp = jnp.concatenate([col, jnp.full((pad,), N, jnp.int32)])
    rowg = rowp.reshape(NGRP, G)
    colg = colp.reshape(NGRP, G)

    zeros16 = jnp.zeros((NP, 8), jnp.float32)
    ones16 = jnp.ones((G, 8), jnp.float32)

    deg16 = _deg_call(colg, zeros16, ones16)
    u, dinv, dinv2 = _pre_call(x, deg16)

    v1 = _prop_call(u, rowg, colg)
    u2 = _mid_call(v1, dinv2)
    w1 = _prop_call(u2, rowg, colg)
    u3 = _mm1_call(w1, dinv, W1, b1.reshape(1, D))
    v2 = _prop_call(u3, rowg, colg)
    u4 = _mid_call(v2, dinv2)
    w2 = _prop_call(u4, rowg, colg)
    return _mm2_call(w2, dinv, W2, b2.reshape(1, D))
